# baseline (speedup 1.0000x reference)
import jax
import jax.numpy as jnp
import numpy as np
from jax.experimental import pallas as pl

_IM = 64
_NN = _IM * _IM
_CD = 64
_QD = 128
_NAG = 16
_ITERS = 32

_OFFS = tuple((di, dj) for di in (-1, 0, 1) for dj in (-1, 0, 1))


def _roll_rows(v, d):
    if d == 0:
        return v
    return jnp.roll(v, -d, axis=0)


def _scene_body(x_ref, wc1_ref, bc1_ref, wc2_ref, bc2_ref,
                w1cat_ref, w2cat_ref, w3cat_ref, b3cat_ref, s0_ref, out_ref):
    f32 = jnp.float32

    pidx = jax.lax.broadcasted_iota(jnp.int32, (_NN, 1), 0)
    i_id = pidx // _IM
    j_id = pidx - i_id * _IM

    masks = []
    for (di, dj) in _OFFS:
        mi = jnp.logical_and(i_id + di >= 0, i_id + di < _IM)
        mj = jnp.logical_and(j_id + dj >= 0, j_id + dj < _IM)
        masks.append(jnp.logical_and(mi, mj).astype(f32))

    def conv3x3(v, w9):
        acc = None
        for dj in (-1, 0, 1):
            vj = _roll_rows(v, dj)
            for di in (-1, 0, 1):
                o = (di + 1) * 3 + (dj + 1)
                xs = _roll_rows(vj, di * _IM) * masks[o]
                t = jax.lax.dot_general(xs, w9[o], (((1,), (1,)), ((), ())),
                                        preferred_element_type=f32)
                acc = t if acc is None else acc + t
        return acc

    def conv1x1(v, w):
        return jax.lax.dot_general(v, w, (((1,), (1,)), ((), ())),
                                   preferred_element_type=f32)

    def bnorm(v):
        m = jnp.mean(v, axis=0, keepdims=True)
        c = v - m
        var = jnp.mean(c * c, axis=0, keepdims=True)
        return c * jax.lax.rsqrt(var + 1e-5)

    x = x_ref[...]
    h = jax.nn.relu(conv3x3(x, wc1_ref[...]) + bc1_ref[...])
    h = jax.nn.relu(conv1x1(h, wc2_ref[...]) + bc2_ref[...])
    y = jax.nn.relu(bnorm(conv3x3(h, w1cat_ref[...])))
    y = bnorm(conv3x3(y, w2cat_ref[...]))
    r = jax.nn.relu(jnp.concatenate([h, h], axis=1) + y)
    f = conv1x1(r, w3cat_ref[...]) + b3cat_ref[...]
    kf = f[:, :_CD]
    qf = f[:, _CD:]

    qn = qf * (1.0 / (jnp.sqrt(jnp.sum(qf * qf, axis=1, keepdims=True)) + 1e-8))
    kn = kf * (1.0 / (jnp.sqrt(jnp.sum(kf * kf, axis=1, keepdims=True)) + 1e-8))

    w9b = []
    for o, (di, dj) in enumerate(_OFFS):
        ks = _roll_rows(kn, di * _IM + dj) * masks[o]
        plane = jnp.sum(qn * ks, axis=1, keepdims=True)
        w9b.append(jnp.broadcast_to(plane, (_NN, _QD)))

    def step(_, s):
        acc = None
        for dj in (-1, 0, 1):
            sj = _roll_rows(s, dj)
            for di in (-1, 0, 1):
                o = (di + 1) * 3 + (dj + 1)
                t = w9b[o] * _roll_rows(sj, di * _IM)
                acc = t if acc is None else acc + t
        ss = jnp.sum(acc * acc, axis=1, keepdims=True)
        return acc * (1.0 / (jnp.sqrt(ss) + 1e-8))

    s = jax.lax.fori_loop(0, _ITERS, step, s0_ref[...])

    rsel = jax.lax.broadcasted_iota(jnp.int32, (_NN, _NAG), 0)
    csel = jax.lax.broadcasted_iota(jnp.int32, (_NN, _NAG), 1)
    sel = (rsel == csel * 273).astype(f32)
    agents = jax.lax.dot_general(sel, s, (((0,), (0,)), ((), ())),
                                 preferred_element_type=f32)
    logits = jax.lax.dot_general(s, agents, (((1,), (1,)), ((), ())),
                                 preferred_element_type=f32)
    lmax = jnp.max(logits, axis=1, keepdims=True)
    e = jnp.exp(logits - lmax)
    out_ref[...] = e / jnp.sum(e, axis=1, keepdims=True)


def kernel(x, Wc1, bc1, Wc2, bc2, Wk1, Wk2, Wk3, bk3, Wq1, Wq2, Wq3,
           init_state, row, col):
    del row, col
    f32 = jnp.float32

    def w3(w):
        return jnp.transpose(w, (2, 3, 0, 1)).reshape(9, w.shape[0], w.shape[1])

    xs = x.reshape(_NN, -1).astype(f32)
    w1cat = jnp.concatenate([w3(Wk1), w3(Wq1)], axis=1)
    w2cat = jnp.zeros((9, 2 * _CD, 2 * _CD), f32)
    w2cat = w2cat.at[:, :_CD, :_CD].set(w3(Wk2)).at[:, _CD:, _CD:].set(w3(Wq2))
    w3cat = jnp.zeros((2 * _CD, 2 * _CD), f32)
    w3cat = w3cat.at[:_CD, :_CD].set(Wk3[:, :, 0, 0]).at[_CD:, _CD:].set(Wq3[:, :, 0, 0])
    b3cat = jnp.concatenate([bk3, jnp.zeros((_CD,), f32)])[None, :]
    args = (
        xs,
        w3(Wc1), bc1[None, :],
        Wc2[:, :, 0, 0], bc2[None, :],
        w1cat, w2cat, w3cat, b3cat,
        init_state.reshape(_NN, _QD),
    )
    masks = pl.pallas_call(
        _scene_body,
        out_shape=jax.ShapeDtypeStruct((_NN, _NAG), f32),
    )(*args)
    return jnp.transpose(masks.reshape(_IM, _IM, _NAG), (2, 0, 1))[None]


# speedup vs baseline: 1.4908x; 1.4908x over previous
import jax
import jax.numpy as jnp
import numpy as np
from jax.experimental import pallas as pl

_IM = 64
_NN = _IM * _IM
_CD = 64
_QD = 128
_NAG = 16
_ITERS = 32

_OFFS = tuple((di, dj) for di in (-1, 0, 1) for dj in (-1, 0, 1))


def _roll_rows(v, d):
    if d == 0:
        return v
    return jnp.roll(v, -d, axis=0)


def _scene_body(x_ref, wc1_ref, bc1_ref, wc2_ref, bc2_ref,
                wk1_ref, wk2_ref, wk3_ref, bk3_ref,
                wq1_ref, wq2_ref, wq3_ref, s0_ref, out_ref):
    f32 = jnp.float32

    pidx = jax.lax.broadcasted_iota(jnp.int32, (_NN, 1), 0)
    i_id = pidx // _IM
    j_id = pidx - i_id * _IM

    masks = []
    for (di, dj) in _OFFS:
        mi = jnp.logical_and(i_id + di >= 0, i_id + di < _IM)
        mj = jnp.logical_and(j_id + dj >= 0, j_id + dj < _IM)
        masks.append(jnp.logical_and(mi, mj).astype(f32))

    def conv3x3(v, w9):
        acc = None
        for dj in (-1, 0, 1):
            vj = _roll_rows(v, dj)
            for di in (-1, 0, 1):
                o = (di + 1) * 3 + (dj + 1)
                xs = _roll_rows(vj, di * _IM) * masks[o]
                t = jax.lax.dot_general(xs, w9[o], (((1,), (1,)), ((), ())),
                                        preferred_element_type=f32)
                acc = t if acc is None else acc + t
        return acc

    def conv1x1(v, w):
        return jax.lax.dot_general(v, w, (((1,), (1,)), ((), ())),
                                   preferred_element_type=f32)

    def bnorm(v):
        m = jnp.mean(v, axis=0, keepdims=True)
        c = v - m
        var = jnp.mean(c * c, axis=0, keepdims=True)
        return c * jax.lax.rsqrt(var + 1e-5)

    def resblock(v, w1, w2):
        y = jax.nn.relu(bnorm(conv3x3(v, w1)))
        y = bnorm(conv3x3(y, w2))
        return jax.nn.relu(v + y)

    x = x_ref[...]
    h = jax.nn.relu(conv3x3(x, wc1_ref[...]) + bc1_ref[...])
    h = jax.nn.relu(conv1x1(h, wc2_ref[...]) + bc2_ref[...])
    kf = conv1x1(resblock(h, wk1_ref[...], wk2_ref[...]), wk3_ref[...]) + bk3_ref[...]
    qf = conv1x1(resblock(h, wq1_ref[...], wq2_ref[...]), wq3_ref[...])

    qn = qf * (1.0 / (jnp.sqrt(jnp.sum(qf * qf, axis=1, keepdims=True)) + 1e-8))
    kn = kf * (1.0 / (jnp.sqrt(jnp.sum(kf * kf, axis=1, keepdims=True)) + 1e-8))

    w9b = []
    for o, (di, dj) in enumerate(_OFFS):
        ks = _roll_rows(kn, di * _IM + dj) * masks[o]
        plane = jnp.sum(qn * ks, axis=1, keepdims=True)
        w9b.append(jnp.broadcast_to(plane, (_NN, _QD)))

    def step(_, s):
        acc = None
        for dj in (-1, 0, 1):
            sj = _roll_rows(s, dj)
            for di in (-1, 0, 1):
                o = (di + 1) * 3 + (dj + 1)
                t = w9b[o] * _roll_rows(sj, di * _IM)
                acc = t if acc is None else acc + t
        ss = jnp.sum(acc * acc, axis=1, keepdims=True)
        return acc * (1.0 / (jnp.sqrt(ss) + 1e-8))

    s = jax.lax.fori_loop(0, _ITERS, step, s0_ref[...], unroll=4)

    rsel = jax.lax.broadcasted_iota(jnp.int32, (_NN, _NAG), 0)
    csel = jax.lax.broadcasted_iota(jnp.int32, (_NN, _NAG), 1)
    sel = (rsel == csel * 273).astype(f32)
    agents = jax.lax.dot_general(sel, s, (((0,), (0,)), ((), ())),
                                 preferred_element_type=f32)
    logits = jax.lax.dot_general(s, agents, (((1,), (1,)), ((), ())),
                                 preferred_element_type=f32)
    lmax = jnp.max(logits, axis=1, keepdims=True)
    e = jnp.exp(logits - lmax)
    out_ref[...] = e / jnp.sum(e, axis=1, keepdims=True)


def kernel(x, Wc1, bc1, Wc2, bc2, Wk1, Wk2, Wk3, bk3, Wq1, Wq2, Wq3,
           init_state, row, col):
    del row, col
    f32 = jnp.float32

    def w3(w):
        return jnp.transpose(w, (2, 3, 0, 1)).reshape(9, w.shape[0], w.shape[1])

    xs = x.reshape(_NN, -1).astype(f32)
    args = (
        xs,
        w3(Wc1), bc1[None, :],
        Wc2[:, :, 0, 0], bc2[None, :],
        w3(Wk1), w3(Wk2), Wk3[:, :, 0, 0], bk3[None, :],
        w3(Wq1), w3(Wq2), Wq3[:, :, 0, 0],
        init_state.reshape(_NN, _QD),
    )
    masks = pl.pallas_call(
        _scene_body,
        out_shape=jax.ShapeDtypeStruct((_NN, _NAG), f32),
    )(*args)
    return jnp.transpose(masks.reshape(_IM, _IM, _NAG), (2, 0, 1))[None]


# speedup vs baseline: 1.5176x; 1.0180x over previous
import jax
import jax.numpy as jnp
import numpy as np
from jax.experimental import pallas as pl

_IM = 64
_NN = _IM * _IM
_CD = 64
_QD = 128
_NAG = 16
_ITERS = 32

_OFFS = tuple((di, dj) for di in (-1, 0, 1) for dj in (-1, 0, 1))


def _roll_rows(v, d):
    if d == 0:
        return v
    return jnp.roll(v, -d, axis=0)


def _scene_body(x_ref, wc1_ref, bc1_ref, wc2_ref, bc2_ref,
                wk1_ref, wk2_ref, wk3_ref, bk3_ref,
                wq1_ref, wq2_ref, wq3_ref, s0_ref, out_ref):
    f32 = jnp.float32

    pidx = jax.lax.broadcasted_iota(jnp.int32, (_NN, 1), 0)
    i_id = pidx // _IM
    j_id = pidx - i_id * _IM

    masks = []
    for (di, dj) in _OFFS:
        mi = jnp.logical_and(i_id + di >= 0, i_id + di < _IM)
        mj = jnp.logical_and(j_id + dj >= 0, j_id + dj < _IM)
        masks.append(jnp.logical_and(mi, mj).astype(f32))

    def conv3x3(v, w9):
        acc = None
        for dj in (-1, 0, 1):
            vj = _roll_rows(v, dj)
            for di in (-1, 0, 1):
                o = (di + 1) * 3 + (dj + 1)
                xs = _roll_rows(vj, di * _IM) * masks[o]
                t = jax.lax.dot_general(xs, w9[o], (((1,), (1,)), ((), ())),
                                        preferred_element_type=f32)
                acc = t if acc is None else acc + t
        return acc

    def conv1x1(v, w):
        return jax.lax.dot_general(v, w, (((1,), (1,)), ((), ())),
                                   preferred_element_type=f32)

    def bnorm(v):
        m = jnp.mean(v, axis=0, keepdims=True)
        c = v - m
        var = jnp.mean(c * c, axis=0, keepdims=True)
        return c * jax.lax.rsqrt(var + 1e-5)

    def resblock(v, w1, w2):
        y = jax.nn.relu(bnorm(conv3x3(v, w1)))
        y = bnorm(conv3x3(y, w2))
        return jax.nn.relu(v + y)

    x = x_ref[...]
    h = jax.nn.relu(conv3x3(x, wc1_ref[...]) + bc1_ref[...])
    h = jax.nn.relu(conv1x1(h, wc2_ref[...]) + bc2_ref[...])
    kf = conv1x1(resblock(h, wk1_ref[...], wk2_ref[...]), wk3_ref[...]) + bk3_ref[...]
    qf = conv1x1(resblock(h, wq1_ref[...], wq2_ref[...]), wq3_ref[...])

    qn = qf * (1.0 / (jnp.sqrt(jnp.sum(qf * qf, axis=1, keepdims=True)) + 1e-8))
    kn = kf * (1.0 / (jnp.sqrt(jnp.sum(kf * kf, axis=1, keepdims=True)) + 1e-8))

    w9b = []
    for o, (di, dj) in enumerate(_OFFS):
        ks = _roll_rows(kn, di * _IM + dj) * masks[o]
        plane = jnp.sum(qn * ks, axis=1, keepdims=True)
        w9b.append(jnp.broadcast_to(plane, (_NN, _QD)))

    def step(_, s):
        acc = None
        for dj in (-1, 0, 1):
            sj = _roll_rows(s, dj)
            for di in (-1, 0, 1):
                o = (di + 1) * 3 + (dj + 1)
                t = w9b[o] * _roll_rows(sj, di * _IM)
                acc = t if acc is None else acc + t
        ss = jnp.sum(acc * acc, axis=1, keepdims=True)
        return acc * (1.0 / (jnp.sqrt(ss) + 1e-8))

    s = jax.lax.fori_loop(0, _ITERS, step, s0_ref[...], unroll=_ITERS)

    rsel = jax.lax.broadcasted_iota(jnp.int32, (_NN, _NAG), 0)
    csel = jax.lax.broadcasted_iota(jnp.int32, (_NN, _NAG), 1)
    sel = (rsel == csel * 273).astype(f32)
    agents = jax.lax.dot_general(sel, s, (((0,), (0,)), ((), ())),
                                 preferred_element_type=f32)
    logits = jax.lax.dot_general(s, agents, (((1,), (1,)), ((), ())),
                                 preferred_element_type=f32)
    lmax = jnp.max(logits, axis=1, keepdims=True)
    e = jnp.exp(logits - lmax)
    out_ref[...] = e / jnp.sum(e, axis=1, keepdims=True)


def kernel(x, Wc1, bc1, Wc2, bc2, Wk1, Wk2, Wk3, bk3, Wq1, Wq2, Wq3,
           init_state, row, col):
    del row, col
    f32 = jnp.float32

    def w3(w):
        return jnp.transpose(w, (2, 3, 0, 1)).reshape(9, w.shape[0], w.shape[1])

    xs = x.reshape(_NN, -1).astype(f32)
    args = (
        xs,
        w3(Wc1), bc1[None, :],
        Wc2[:, :, 0, 0], bc2[None, :],
        w3(Wk1), w3(Wk2), Wk3[:, :, 0, 0], bk3[None, :],
        w3(Wq1), w3(Wq2), Wq3[:, :, 0, 0],
        init_state.reshape(_NN, _QD),
    )
    masks = pl.pallas_call(
        _scene_body,
        out_shape=jax.ShapeDtypeStruct((_NN, _NAG), f32),
    )(*args)
    return jnp.transpose(masks.reshape(_IM, _IM, _NAG), (2, 0, 1))[None]


# speedup vs baseline: 1.5528x; 1.0232x over previous
import jax
import jax.numpy as jnp
import numpy as np
from jax.experimental import pallas as pl
from jax.experimental.pallas import tpu as pltpu

_IM = 64
_NN = _IM * _IM
_CD = 64
_QD = 128
_NAG = 16
_ITERS = 32

_OFFS = tuple((di, dj) for di in (-1, 0, 1) for dj in (-1, 0, 1))


def _roll_rows(v, d):
    if d == 0:
        return v
    return jnp.roll(v, -d, axis=0)


def _scene_body(x_ref, wc1_ref, bc1_ref, wc2_ref, bc2_ref,
                wk1_ref, wk2_ref, wk3_ref, bk3_ref,
                wq1_ref, wq2_ref, wq3_ref, s0_ref, out_ref, pad_ref):
    f32 = jnp.float32

    pidx = jax.lax.broadcasted_iota(jnp.int32, (_NN, 1), 0)
    i_id = pidx // _IM
    j_id = pidx - i_id * _IM

    masks = []
    for (di, dj) in _OFFS:
        mi = jnp.logical_and(i_id + di >= 0, i_id + di < _IM)
        mj = jnp.logical_and(j_id + dj >= 0, j_id + dj < _IM)
        masks.append(jnp.logical_and(mi, mj).astype(f32))

    def conv3x3(v, w9):
        acc = None
        for dj in (-1, 0, 1):
            vj = _roll_rows(v, dj)
            for di in (-1, 0, 1):
                o = (di + 1) * 3 + (dj + 1)
                xs = _roll_rows(vj, di * _IM) * masks[o]
                t = jax.lax.dot_general(xs, w9[o], (((1,), (1,)), ((), ())),
                                        preferred_element_type=f32)
                acc = t if acc is None else acc + t
        return acc

    def conv1x1(v, w):
        return jax.lax.dot_general(v, w, (((1,), (1,)), ((), ())),
                                   preferred_element_type=f32)

    def bnorm(v):
        m = jnp.mean(v, axis=0, keepdims=True)
        c = v - m
        var = jnp.mean(c * c, axis=0, keepdims=True)
        return c * jax.lax.rsqrt(var + 1e-5)

    def resblock(v, w1, w2):
        y = jax.nn.relu(bnorm(conv3x3(v, w1)))
        y = bnorm(conv3x3(y, w2))
        return jax.nn.relu(v + y)

    x = x_ref[...]
    h = jax.nn.relu(conv3x3(x, wc1_ref[...]) + bc1_ref[...])
    h = jax.nn.relu(conv1x1(h, wc2_ref[...]) + bc2_ref[...])
    kf = conv1x1(resblock(h, wk1_ref[...], wk2_ref[...]), wk3_ref[...]) + bk3_ref[...]
    qf = conv1x1(resblock(h, wq1_ref[...], wq2_ref[...]), wq3_ref[...])

    qn = qf * (1.0 / (jnp.sqrt(jnp.sum(qf * qf, axis=1, keepdims=True)) + 1e-8))
    kn = kf * (1.0 / (jnp.sqrt(jnp.sum(kf * kf, axis=1, keepdims=True)) + 1e-8))

    w9b = []
    for o, (di, dj) in enumerate(_OFFS):
        ks = _roll_rows(kn, di * _IM + dj) * masks[o]
        plane = jnp.sum(qn * ks, axis=1, keepdims=True)
        w9b.append(jnp.broadcast_to(plane, (_NN, _QD)))

    pad = _IM + 8
    pad_ref[0:pad, :] = jnp.zeros((pad, _QD), f32)
    pad_ref[pad + _NN:, :] = jnp.zeros((pad, _QD), f32)

    s = s0_ref[...]
    for _ in range(_ITERS):
        pad_ref[pad:pad + _NN, :] = s
        acc = None
        for o, (di, dj) in enumerate(_OFFS):
            base = pad + di * _IM + dj
            t = w9b[o] * pad_ref[base:base + _NN, :]
            acc = t if acc is None else acc + t
        ss = jnp.sum(acc * acc, axis=1, keepdims=True)
        s = acc * (1.0 / (jnp.sqrt(ss) + 1e-8))

    rsel = jax.lax.broadcasted_iota(jnp.int32, (_NN, _NAG), 0)
    csel = jax.lax.broadcasted_iota(jnp.int32, (_NN, _NAG), 1)
    sel = (rsel == csel * 273).astype(f32)
    agents = jax.lax.dot_general(sel, s, (((0,), (0,)), ((), ())),
                                 preferred_element_type=f32)
    logits = jax.lax.dot_general(s, agents, (((1,), (1,)), ((), ())),
                                 preferred_element_type=f32)
    lmax = jnp.max(logits, axis=1, keepdims=True)
    e = jnp.exp(logits - lmax)
    out_ref[...] = e / jnp.sum(e, axis=1, keepdims=True)


def kernel(x, Wc1, bc1, Wc2, bc2, Wk1, Wk2, Wk3, bk3, Wq1, Wq2, Wq3,
           init_state, row, col):
    del row, col
    f32 = jnp.float32

    def w3(w):
        return jnp.transpose(w, (2, 3, 0, 1)).reshape(9, w.shape[0], w.shape[1])

    xs = x.reshape(_NN, -1).astype(f32)
    args = (
        xs,
        w3(Wc1), bc1[None, :],
        Wc2[:, :, 0, 0], bc2[None, :],
        w3(Wk1), w3(Wk2), Wk3[:, :, 0, 0], bk3[None, :],
        w3(Wq1), w3(Wq2), Wq3[:, :, 0, 0],
        init_state.reshape(_NN, _QD),
    )
    masks = pl.pallas_call(
        _scene_body,
        out_shape=jax.ShapeDtypeStruct((_NN, _NAG), f32),
        scratch_shapes=[pltpu.VMEM((_NN + 2 * (_IM + 8), _QD), f32)],
    )(*args)
    return jnp.transpose(masks.reshape(_IM, _IM, _NAG), (2, 0, 1))[None]


# speedup vs baseline: 1.5542x; 1.0009x over previous
import jax
import jax.numpy as jnp
import numpy as np
from jax.experimental import pallas as pl
from jax.experimental.pallas import tpu as pltpu

_IM = 64
_NN = _IM * _IM
_CD = 64
_QD = 128
_NAG = 16
_ITERS = 32

_OFFS = tuple((di, dj) for di in (-1, 0, 1) for dj in (-1, 0, 1))


def _roll_rows(v, d):
    if d == 0:
        return v
    return jnp.roll(v, -d, axis=0)


def _scene_body(x_ref, wc1_ref, bc1_ref, wc2_ref, bc2_ref,
                wk1_ref, wk2_ref, wk3_ref, bk3_ref,
                wq1_ref, wq2_ref, wq3_ref, s0_ref, out_ref, pad_ref):
    f32 = jnp.float32

    pidx = jax.lax.broadcasted_iota(jnp.int32, (_NN, 1), 0)
    i_id = pidx // _IM
    j_id = pidx - i_id * _IM

    masks = []
    for (di, dj) in _OFFS:
        mi = jnp.logical_and(i_id + di >= 0, i_id + di < _IM)
        mj = jnp.logical_and(j_id + dj >= 0, j_id + dj < _IM)
        masks.append(jnp.logical_and(mi, mj).astype(f32))

    def conv3x3(v, w9):
        acc = None
        for dj in (-1, 0, 1):
            vj = _roll_rows(v, dj)
            for di in (-1, 0, 1):
                o = (di + 1) * 3 + (dj + 1)
                xs = _roll_rows(vj, di * _IM) * masks[o]
                t = jax.lax.dot_general(xs, w9[o], (((1,), (1,)), ((), ())),
                                        preferred_element_type=f32)
                acc = t if acc is None else acc + t
        return acc

    def conv1x1(v, w):
        return jax.lax.dot_general(v, w, (((1,), (1,)), ((), ())),
                                   preferred_element_type=f32)

    def bnorm(v):
        m = jnp.mean(v, axis=0, keepdims=True)
        c = v - m
        var = jnp.mean(c * c, axis=0, keepdims=True)
        return c * jax.lax.rsqrt(var + 1e-5)

    def resblock(v, w1, w2):
        y = jax.nn.relu(bnorm(conv3x3(v, w1)))
        y = bnorm(conv3x3(y, w2))
        return jax.nn.relu(v + y)

    x = x_ref[...]
    h = jax.nn.relu(conv3x3(x, wc1_ref[...]) + bc1_ref[...])
    h = jax.nn.relu(conv1x1(h, wc2_ref[...]) + bc2_ref[...])
    kf = conv1x1(resblock(h, wk1_ref[...], wk2_ref[...]), wk3_ref[...]) + bk3_ref[...]
    qf = conv1x1(resblock(h, wq1_ref[...], wq2_ref[...]), wq3_ref[...])

    qn = qf * (1.0 / (jnp.sqrt(jnp.sum(qf * qf, axis=1, keepdims=True)) + 1e-8))
    kn = kf * (1.0 / (jnp.sqrt(jnp.sum(kf * kf, axis=1, keepdims=True)) + 1e-8))

    w9b = []
    for o, (di, dj) in enumerate(_OFFS):
        ks = _roll_rows(kn, di * _IM + dj) * masks[o]
        plane = jnp.sum(qn * ks, axis=1, keepdims=True)
        w9b.append(jnp.broadcast_to(plane, (_NN, _QD)))

    pad = _IM + 8
    pad_ref[0:pad, :] = jnp.zeros((pad, _QD), f32)
    pad_ref[pad + _NN:, :] = jnp.zeros((pad, _QD), f32)

    pad_ref[pad:pad + _NN, :] = s0_ref[...]
    for _ in range(_ITERS):
        acc = None
        for o, (di, dj) in enumerate(_OFFS):
            base = pad + di * _IM + dj
            t = w9b[o] * pad_ref[base:base + _NN, :]
            acc = t if acc is None else acc + t
        ss = jnp.sum(acc * acc, axis=1, keepdims=True)
        pad_ref[pad:pad + _NN, :] = acc * (1.0 / (jnp.sqrt(ss) + 1e-8))
    s = pad_ref[pad:pad + _NN, :]

    rsel = jax.lax.broadcasted_iota(jnp.int32, (_NN, _NAG), 0)
    csel = jax.lax.broadcasted_iota(jnp.int32, (_NN, _NAG), 1)
    sel = (rsel == csel * 273).astype(f32)
    agents = jax.lax.dot_general(sel, s, (((0,), (0,)), ((), ())),
                                 preferred_element_type=f32)
    logits = jax.lax.dot_general(s, agents, (((1,), (1,)), ((), ())),
                                 preferred_element_type=f32)
    lmax = jnp.max(logits, axis=1, keepdims=True)
    e = jnp.exp(logits - lmax)
    out_ref[...] = e / jnp.sum(e, axis=1, keepdims=True)


def kernel(x, Wc1, bc1, Wc2, bc2, Wk1, Wk2, Wk3, bk3, Wq1, Wq2, Wq3,
           init_state, row, col):
    del row, col
    f32 = jnp.float32

    def w3(w):
        return jnp.transpose(w, (2, 3, 0, 1)).reshape(9, w.shape[0], w.shape[1])

    xs = x.reshape(_NN, -1).astype(f32)
    args = (
        xs,
        w3(Wc1), bc1[None, :],
        Wc2[:, :, 0, 0], bc2[None, :],
        w3(Wk1), w3(Wk2), Wk3[:, :, 0, 0], bk3[None, :],
        w3(Wq1), w3(Wq2), Wq3[:, :, 0, 0],
        init_state.reshape(_NN, _QD),
    )
    masks = pl.pallas_call(
        _scene_body,
        out_shape=jax.ShapeDtypeStruct((_NN, _NAG), f32),
        scratch_shapes=[pltpu.VMEM((_NN + 2 * (_IM + 8), _QD), f32)],
    )(*args)
    return jnp.transpose(masks.reshape(_IM, _IM, _NAG), (2, 0, 1))[None]
